# Initial kernel scaffold; baseline (speedup 1.0000x reference)
import jax
import jax.numpy as jnp
from jax.experimental import pallas as pl


def kernel(x, edge_index, edge_attr, batch, W1, a_src1, a_dst1, b1, W2, a_src2, a_dst2, b2, Wl):
    raise NotImplementedError("write your pallas kernel here")


# speedup vs baseline: 73.3595x; 73.3595x over previous
import functools

import jax
import jax.numpy as jnp
from jax import lax
from jax.experimental import pallas as pl
from jax.experimental.pallas import tpu as pltpu
from jax.experimental.pallas import tpu_sc as plsc

N = 100000
E = 1600000
H = 32
HH = 16
NC = 2
NS = 16
NW = NC * NS
N_PAD = 100096
E_PAD = 1605632
ER = E_PAD // 128
SL = N_PAD // NS

W_ROWS = ER // NW
S_ROWS = ER // NS
WB_CH = 184
MCH = 4

_mesh = functools.partial(
    plsc.VectorSubcoreMesh, core_axis_name="c", subcore_axis_name="s",
    num_cores=NC, num_subcores=NS)


def _zero16():
  return jnp.zeros((16,), jnp.float32)


def _stage_slice(sid):
  return pl.ds(sid * SL, SL)


NR8 = N_PAD // 8
NR128 = N_PAD // 128
MBLK = NR8 // 4


def _tc_pre_body(xp_ref, kh0_ref, kh1_ref, kva_ref, kvd_ref,
                 hp_ref, as_ref, ad_ref):
  xp = xp_ref[...]
  hp_ref[0] = jnp.dot(xp, kh0_ref[...], preferred_element_type=jnp.float32)
  hp_ref[1] = jnp.dot(xp, kh1_ref[...], preferred_element_type=jnp.float32)
  as_ref[...] = jnp.dot(xp, kva_ref[...], preferred_element_type=jnp.float32)
  ad_ref[...] = jnp.dot(xp, kvd_ref[...], preferred_element_type=jnp.float32)


def _tc_pre(xp, Kh0, Kh1, Kva, Kvd):
  return pl.pallas_call(
      _tc_pre_body,
      out_shape=[
          jax.ShapeDtypeStruct((2, NR8, 128), jnp.float32),
          jax.ShapeDtypeStruct((NR8, 8), jnp.float32),
          jax.ShapeDtypeStruct((NR8, 8), jnp.float32),
      ],
  )(xp, Kh0, Kh1, Kva, Kvd)


def _tc_mid_body(o1_ref, b1p_ref, kw_ref, a2_ref,
                 h2_ref, as2_ref, ad2_ref):
  pid = pl.program_id(0)
  x0 = o1_ref[0] + b1p_ref[0]
  x1 = o1_ref[1] + b1p_ref[1]
  e0 = jnp.where(x0 > 0, x0, jnp.exp(jnp.minimum(x0, 0.0)) - 1.0)
  e1 = jnp.where(x1 > 0, x1, jnp.exp(jnp.minimum(x1, 0.0)) - 1.0)
  h2 = (jnp.dot(e0, kw_ref[0], preferred_element_type=jnp.float32)
        + jnp.dot(e1, kw_ref[1], preferred_element_type=jnp.float32))
  node = (pid * (MBLK * 8)
          + 8 * lax.broadcasted_iota(jnp.int32, (MBLK, 8), 0)
          + lax.broadcasted_iota(jnp.int32, (MBLK, 8), 1))
  h2 = jnp.where(node < N, h2, 0.0)
  h2_ref[...] = h2
  as2_ref[...] = h2 * a2_ref[0, 0]
  ad2_ref[...] = h2 * a2_ref[0, 1]


def _tc_mid(o1p, b1p, KW, a2):
  return pl.pallas_call(
      _tc_mid_body,
      grid=(NR8 // MBLK,),
      in_specs=[
          pl.BlockSpec((2, MBLK, 128), lambda i: (0, i, 0)),
          pl.BlockSpec((2, 128), lambda i: (0, 0)),
          pl.BlockSpec((2, 128, 8), lambda i: (0, 0, 0)),
          pl.BlockSpec((1, 2), lambda i: (0, 0)),
      ],
      out_specs=[
          pl.BlockSpec((MBLK, 8), lambda i: (i, 0)),
          pl.BlockSpec((MBLK, 8), lambda i: (i, 0)),
          pl.BlockSpec((MBLK, 8), lambda i: (i, 0)),
      ],
      out_shape=[
          jax.ShapeDtypeStruct((NR8, 8), jnp.float32),
          jax.ShapeDtypeStruct((NR8, 8), jnp.float32),
          jax.ShapeDtypeStruct((NR8, 8), jnp.float32),
      ],
  )(o1p, b1p, KW, a2)


def _tc_post_body(o2_ref, pp_ref, out_ref):
  s = (o2_ref[0] + o2_ref[1] + pp_ref[0, 0]) * pp_ref[0, 1]
  out_ref[...] = jax.nn.sigmoid(s)


def _tc_post(o2p, pp):
  return pl.pallas_call(
      _tc_post_body,
      out_shape=jax.ShapeDtypeStruct((NR128, 128), jnp.float32),
  )(o2p, pp)


def _sc_att_body(src_ref, dst_ref, as_ref, ad_ref,
                 den_ref, ex_ref,
                 as_sh, ad_sh, den_sh,
                 src_v, dst_v, asg_v, adg_v, ex_v,
                 zb_v, sem, sem2):
  cid = lax.axis_index("c")
  sid = lax.axis_index("s")
  wid = sid * NC + cid
  sl = _stage_slice(sid)

  pltpu.sync_copy(as_ref.at[sl], zb_v)
  pltpu.sync_copy(zb_v, as_sh.at[sl])
  pltpu.sync_copy(ad_ref.at[sl], zb_v)
  pltpu.sync_copy(zb_v, ad_sh.at[sl])

  def _z(i, _):
    zb_v[pl.ds(i * 16, 16)] = _zero16()
    return 0
  lax.fori_loop(0, SL // 16, _z, 0)
  pltpu.sync_copy(zb_v, den_sh.at[sl])
  plsc.subcore_barrier()

  def _chunk(t, _):
    rbase = wid * W_ROWS + t * 8
    pltpu.sync_copy(src_ref.at[pl.ds(rbase, 8)], src_v)
    pltpu.sync_copy(dst_ref.at[pl.ds(rbase, 8)], dst_v)
    waits = []
    for i in range(8):
      waits.append(pltpu.async_copy(as_sh.at[src_v.at[i]], asg_v.at[i], sem))
      waits.append(pltpu.async_copy(ad_sh.at[dst_v.at[i]], adg_v.at[i], sem))
    for w in waits:
      w.wait()
    for r in range(8):
      for g in range(8):
        o = g * 16
        e = asg_v[r, pl.ds(o, 16)] + adg_v[r, pl.ds(o, 16)]
        e = jnp.where(e >= 0.0, e, 0.2 * e)
        ex_v[r, pl.ds(o, 16)] = jnp.exp(e)
    pltpu.sync_copy(ex_v, ex_ref.at[pl.ds(rbase, 8)])
    waits = []
    for i in range(8):
      waits.append(pltpu.async_copy(ex_v.at[i], den_sh.at[dst_v.at[i]], sem2,
                                    add=True))
    for w in waits:
      w.wait()
    return 0

  lax.fori_loop(0, W_ROWS // 8, _chunk, 0)
  plsc.subcore_barrier()
  pltpu.sync_copy(den_sh.at[sl], zb_v)
  pltpu.sync_copy(zb_v, den_ref.at[pl.ds(cid * N_PAD + sid * SL, SL)])


def _sc_att(src2, dst2, asp, adp):
  f = pl.kernel(
      _sc_att_body,
      out_type=[
          jax.ShapeDtypeStruct((NC * N_PAD,), jnp.float32),
          jax.ShapeDtypeStruct((ER, 128), jnp.float32),
      ],
      mesh=_mesh(),
      compiler_params=pltpu.CompilerParams(use_tc_tiling_on_sc=False),
      scratch_types=[
          pltpu.VMEM_SHARED((N_PAD,), jnp.float32),
          pltpu.VMEM_SHARED((N_PAD,), jnp.float32),
          pltpu.VMEM_SHARED((N_PAD,), jnp.float32),
          pltpu.VMEM((8, 128), jnp.int32),
          pltpu.VMEM((8, 128), jnp.int32),
          pltpu.VMEM((8, 128), jnp.float32),
          pltpu.VMEM((8, 128), jnp.float32),
          pltpu.VMEM((8, 128), jnp.float32),
          pltpu.VMEM((SL,), jnp.float32),
          pltpu.SemaphoreType.DMA,
          pltpu.SemaphoreType.DMA,
      ],
  )
  return f(src2, dst2, asp, adp)


def _sc_msg32_body(src_ref, dst_ref, ex_ref, ew_ref, denp_ref, h1s_ref,
                   out_ref,
                   den_sh, acc_sh,
                   src_v, dst_v, ex_v, ew_v, dg_v, w_v, rows_v,
                   pa_v, pb_v, zwb_v, sem, sem2, sem3):
  cid = lax.axis_index("c")
  sid = lax.axis_index("s")

  def _dcomb(k, _):
    off = sid * SL + k * WB_CH
    pltpu.sync_copy(denp_ref.at[pl.ds(off, WB_CH)], pa_v)
    pltpu.sync_copy(denp_ref.at[pl.ds(N_PAD + off, WB_CH)], pb_v)

    def _add(i, _):
      o = pl.ds(i * 16, 16)
      pa_v[o] = pa_v[o] + pb_v[o]
      return 0
    lax.fori_loop(0, WB_CH // 16, _add, 0)
    pltpu.sync_copy(pa_v, den_sh.at[pl.ds(off, WB_CH)])
    return 0
  lax.fori_loop(0, SL // WB_CH, _dcomb, 0)

  def _z(i, _):
    zwb_v[i] = _zero16()
    return 0
  lax.fori_loop(0, WB_CH, _z, 0)

  def _zc(k, _):
    pltpu.sync_copy(zwb_v, acc_sh.at[pl.ds(sid * SL + k * WB_CH, WB_CH), :])
    return 0
  lax.fori_loop(0, SL // WB_CH, _zc, 0)
  plsc.subcore_barrier()

  h_half = h1s_ref.at[cid]

  def _chunk(t, _):
    rbase = sid * S_ROWS + t * MCH
    pltpu.sync_copy(src_ref.at[pl.ds(rbase, MCH)], src_v)
    pltpu.sync_copy(dst_ref.at[pl.ds(rbase, MCH)], dst_v)
    pltpu.sync_copy(ex_ref.at[pl.ds(rbase, MCH)], ex_v)
    pltpu.sync_copy(ew_ref.at[pl.ds(rbase, MCH)], ew_v)
    gw, rw = [], []
    for i in range(MCH):
      gw.append(pltpu.async_copy(den_sh.at[dst_v.at[i]], dg_v.at[i], sem))
      rw.append(pltpu.async_copy(h_half.at[src_v.at[i]], rows_v.at[i], sem2))
    for w in gw:
      w.wait()

    def _wrow(r, _):
      for g in range(8):
        o = pl.ds(g * 16, 16)
        w_v[r, o] = ex_v[r, o] / (dg_v[r, o] + 1e-16) * ew_v[r, o]
      return 0
    lax.fori_loop(0, MCH, _wrow, 0)
    for w in rw:
      w.wait()

    def _scale_r(r, _):
      def _scale_c(g, _):
        wg = w_v[r, pl.ds(g * 16, 16)]
        for u in range(16):
          c = g * 16 + u
          rows_v[r, c] = rows_v[r, c] * jnp.broadcast_to(wg[u], (16,))
        return 0
      lax.fori_loop(0, 8, _scale_c, 0)
      return 0
    lax.fori_loop(0, MCH, _scale_r, 0)

    sw = []
    for i in range(MCH):
      sw.append(pltpu.async_copy(rows_v.at[i], acc_sh.at[dst_v.at[i]], sem3,
                                 add=True))
    for w in sw:
      w.wait()
    return 0

  lax.fori_loop(0, S_ROWS // MCH, _chunk, 0)
  plsc.subcore_barrier()

  def _wb(k, _):
    ro = sid * SL + k * WB_CH
    pltpu.sync_copy(acc_sh.at[pl.ds(ro, WB_CH), :], zwb_v)
    pltpu.sync_copy(zwb_v, out_ref.at[cid, pl.ds(ro, WB_CH), :])
    return 0
  lax.fori_loop(0, SL // WB_CH, _wb, 0)


def _sc_msg32(src2, dst2, ex2, ew2, den_part, h1s):
  f = pl.kernel(
      _sc_msg32_body,
      out_type=jax.ShapeDtypeStruct((2, N_PAD, HH), jnp.float32),
      mesh=_mesh(),
      compiler_params=pltpu.CompilerParams(use_tc_tiling_on_sc=False),
      scratch_types=[
          pltpu.VMEM_SHARED((N_PAD,), jnp.float32),
          pltpu.VMEM_SHARED((N_PAD, HH), jnp.float32),
          pltpu.VMEM((MCH, 128), jnp.int32),
          pltpu.VMEM((MCH, 128), jnp.int32),
          pltpu.VMEM((MCH, 128), jnp.float32),
          pltpu.VMEM((MCH, 128), jnp.float32),
          pltpu.VMEM((MCH, 128), jnp.float32),
          pltpu.VMEM((MCH, 128), jnp.float32),
          pltpu.VMEM((MCH, 128, HH), jnp.float32),
          pltpu.VMEM((WB_CH,), jnp.float32),
          pltpu.VMEM((WB_CH,), jnp.float32),
          pltpu.VMEM((WB_CH, HH), jnp.float32),
          pltpu.SemaphoreType.DMA,
          pltpu.SemaphoreType.DMA,
          pltpu.SemaphoreType.DMA,
      ],
  )
  return f(src2, dst2, ex2, ew2, den_part, h1s)


def _sc_msg1_body(src_ref, dst_ref, ex_ref, ew_ref, denp_ref, h2_ref,
                  out_ref,
                  den_sh, h2_sh, o2_sh,
                  src_v, dst_v, ex_v, ew_v, dg_v, hg_v, v_v,
                  pa_v, pb_v, zb_v, sem, sem2, sem3):
  cid = lax.axis_index("c")
  sid = lax.axis_index("s")
  wid = sid * NC + cid
  sl = _stage_slice(sid)

  pltpu.sync_copy(denp_ref.at[pl.ds(sid * SL, SL)], pa_v)
  pltpu.sync_copy(denp_ref.at[pl.ds(N_PAD + sid * SL, SL)], pb_v)

  def _add(i, _):
    o = pl.ds(i * 16, 16)
    pa_v[o] = pa_v[o] + pb_v[o]
    return 0
  lax.fori_loop(0, SL // 16, _add, 0)
  pltpu.sync_copy(pa_v, den_sh.at[sl])
  pltpu.sync_copy(h2_ref.at[sl], pb_v)
  pltpu.sync_copy(pb_v, h2_sh.at[sl])

  def _z(i, _):
    zb_v[pl.ds(i * 16, 16)] = _zero16()
    return 0
  lax.fori_loop(0, SL // 16, _z, 0)
  pltpu.sync_copy(zb_v, o2_sh.at[sl])
  plsc.subcore_barrier()

  def _chunk(t, _):
    rbase = wid * W_ROWS + t * 8
    pltpu.sync_copy(src_ref.at[pl.ds(rbase, 8)], src_v)
    pltpu.sync_copy(dst_ref.at[pl.ds(rbase, 8)], dst_v)
    pltpu.sync_copy(ex_ref.at[pl.ds(rbase, 8)], ex_v)
    pltpu.sync_copy(ew_ref.at[pl.ds(rbase, 8)], ew_v)
    waits = []
    for i in range(8):
      waits.append(pltpu.async_copy(den_sh.at[dst_v.at[i]], dg_v.at[i], sem))
      waits.append(pltpu.async_copy(h2_sh.at[src_v.at[i]], hg_v.at[i], sem2))
    for w in waits:
      w.wait()
    for r in range(8):
      for g in range(8):
        o = pl.ds(g * 16, 16)
        v_v[r, o] = ex_v[r, o] / (dg_v[r, o] + 1e-16) * ew_v[r, o] * hg_v[r, o]
    waits = []
    for i in range(8):
      waits.append(pltpu.async_copy(v_v.at[i], o2_sh.at[dst_v.at[i]], sem3,
                                    add=True))
    for w in waits:
      w.wait()
    return 0

  lax.fori_loop(0, W_ROWS // 8, _chunk, 0)
  plsc.subcore_barrier()
  pltpu.sync_copy(o2_sh.at[sl], pa_v)
  pltpu.sync_copy(pa_v, out_ref.at[pl.ds(cid * N_PAD + sid * SL, SL)])


def _sc_msg1(src2, dst2, ex2, ew2, den_part, h2p):
  f = pl.kernel(
      _sc_msg1_body,
      out_type=jax.ShapeDtypeStruct((NC * N_PAD,), jnp.float32),
      mesh=_mesh(),
      compiler_params=pltpu.CompilerParams(use_tc_tiling_on_sc=False),
      scratch_types=[
          pltpu.VMEM_SHARED((N_PAD,), jnp.float32),
          pltpu.VMEM_SHARED((N_PAD,), jnp.float32),
          pltpu.VMEM_SHARED((N_PAD,), jnp.float32),
          pltpu.VMEM((8, 128), jnp.int32),
          pltpu.VMEM((8, 128), jnp.int32),
          pltpu.VMEM((8, 128), jnp.float32),
          pltpu.VMEM((8, 128), jnp.float32),
          pltpu.VMEM((8, 128), jnp.float32),
          pltpu.VMEM((8, 128), jnp.float32),
          pltpu.VMEM((8, 128), jnp.float32),
          pltpu.VMEM((SL,), jnp.float32),
          pltpu.VMEM((SL,), jnp.float32),
          pltpu.VMEM((SL,), jnp.float32),
          pltpu.SemaphoreType.DMA,
          pltpu.SemaphoreType.DMA,
          pltpu.SemaphoreType.DMA,
      ],
  )
  return f(src2, dst2, ex2, ew2, den_part, h2p)


def kernel(x, edge_index, edge_attr, batch, W1, a_src1, a_dst1, b1,
           W2, a_src2, a_dst2, b2, Wl):
  del batch
  src = edge_index[0]
  dst = edge_index[1]
  ew = edge_attr[:, 0]

  npad = E_PAD - E
  spread = jnp.arange(npad, dtype=jnp.int32) % 8
  src2 = jnp.concatenate([src, spread]).reshape(ER, 128)
  dst2 = jnp.concatenate([dst, N + spread]).reshape(ER, 128)
  ew2 = jnp.concatenate([ew, jnp.zeros((npad,), jnp.float32)]).reshape(ER, 128)

  f32 = jnp.float32
  eye8 = jnp.eye(8, dtype=f32)
  xp = jnp.concatenate([x, jnp.zeros((N_PAD - N, x.shape[1]), f32)])
  xp = xp.reshape(NR8, 8 * x.shape[1])
  Kh0 = jnp.kron(eye8, W1[:, :HH])
  Kh1 = jnp.kron(eye8, W1[:, HH:])
  Kva = jnp.kron(eye8, (W1 @ a_src1)[:, None])
  Kvd = jnp.kron(eye8, (W1 @ a_dst1)[:, None])

  hpack, aspe, adpe = _tc_pre(xp, Kh0, Kh1, Kva, Kvd)
  h1s = hpack.reshape(2, N_PAD, HH)
  den1, ex1 = _sc_att(src2, dst2, aspe.reshape(N_PAD), adpe.reshape(N_PAD))
  out1s = _sc_msg32(src2, dst2, ex1, ew2, den1, h1s)

  o1p = out1s.reshape(2, NR8, 128)
  b1p = jnp.stack([jnp.tile(b1[:HH], 8), jnp.tile(b1[HH:], 8)])
  KW = jnp.stack([jnp.kron(eye8, W2[:HH]), jnp.kron(eye8, W2[HH:])])
  a2 = jnp.concatenate([a_src2, a_dst2]).reshape(1, 2)
  h2e, as2e, ad2e = _tc_mid(o1p, b1p, KW, a2)

  den2, ex2 = _sc_att(src2, dst2, as2e.reshape(N_PAD), ad2e.reshape(N_PAD))
  out2p = _sc_msg1(src2, dst2, ex2, ew2, den2, h2e.reshape(N_PAD))

  o2p = out2p.reshape(2, NR128, 128)
  pp = jnp.concatenate([b2, Wl[0]]).reshape(1, 2)
  y = _tc_post(o2p, pp)
  return y.reshape(N_PAD)[:N, None]


# speedup vs baseline: 84.5044x; 1.1519x over previous
import functools

import jax
import jax.numpy as jnp
from jax import lax
from jax.experimental import pallas as pl
from jax.experimental.pallas import tpu as pltpu
from jax.experimental.pallas import tpu_sc as plsc

N = 100000
E = 1600000
H = 32
HH = 16
NC = 2
NS = 16
NW = NC * NS
N_PAD = 100096
E_PAD = 1605632
ER = E_PAD // 128
SL = N_PAD // NS

W_ROWS = ER // NW
S_ROWS = ER // NS
WB_CH = 184
MCH = 7

_mesh = functools.partial(
    plsc.VectorSubcoreMesh, core_axis_name="c", subcore_axis_name="s",
    num_cores=NC, num_subcores=NS)


def _zero16():
  return jnp.zeros((16,), jnp.float32)


def _stage_slice(sid):
  return pl.ds(sid * SL, SL)


NR8 = N_PAD // 8
NR128 = N_PAD // 128
MBLK = NR8 // 4


def _tc_pre_body(xp_ref, kh0_ref, kh1_ref, kva_ref, kvd_ref,
                 hp_ref, as_ref, ad_ref):
  xp = xp_ref[...]
  hp_ref[0] = jnp.dot(xp, kh0_ref[...], preferred_element_type=jnp.float32)
  hp_ref[1] = jnp.dot(xp, kh1_ref[...], preferred_element_type=jnp.float32)
  as_ref[...] = jnp.dot(xp, kva_ref[...], preferred_element_type=jnp.float32)
  ad_ref[...] = jnp.dot(xp, kvd_ref[...], preferred_element_type=jnp.float32)


def _tc_pre(xp, Kh0, Kh1, Kva, Kvd):
  return pl.pallas_call(
      _tc_pre_body,
      out_shape=[
          jax.ShapeDtypeStruct((2, NR8, 128), jnp.float32),
          jax.ShapeDtypeStruct((NR8, 8), jnp.float32),
          jax.ShapeDtypeStruct((NR8, 8), jnp.float32),
      ],
  )(xp, Kh0, Kh1, Kva, Kvd)


def _tc_mid_body(o1_ref, b1p_ref, kw_ref, a2_ref,
                 h2_ref, as2_ref, ad2_ref):
  pid = pl.program_id(0)
  x0 = o1_ref[0] + b1p_ref[0]
  x1 = o1_ref[1] + b1p_ref[1]
  e0 = jnp.where(x0 > 0, x0, jnp.exp(jnp.minimum(x0, 0.0)) - 1.0)
  e1 = jnp.where(x1 > 0, x1, jnp.exp(jnp.minimum(x1, 0.0)) - 1.0)
  h2 = (jnp.dot(e0, kw_ref[0], preferred_element_type=jnp.float32)
        + jnp.dot(e1, kw_ref[1], preferred_element_type=jnp.float32))
  node = (pid * (MBLK * 8)
          + 8 * lax.broadcasted_iota(jnp.int32, (MBLK, 8), 0)
          + lax.broadcasted_iota(jnp.int32, (MBLK, 8), 1))
  h2 = jnp.where(node < N, h2, 0.0)
  h2_ref[...] = h2
  as2_ref[...] = h2 * a2_ref[0, 0]
  ad2_ref[...] = h2 * a2_ref[0, 1]


def _tc_mid(o1p, b1p, KW, a2):
  return pl.pallas_call(
      _tc_mid_body,
      grid=(NR8 // MBLK,),
      in_specs=[
          pl.BlockSpec((2, MBLK, 128), lambda i: (0, i, 0)),
          pl.BlockSpec((2, 128), lambda i: (0, 0)),
          pl.BlockSpec((2, 128, 8), lambda i: (0, 0, 0)),
          pl.BlockSpec((1, 2), lambda i: (0, 0)),
      ],
      out_specs=[
          pl.BlockSpec((MBLK, 8), lambda i: (i, 0)),
          pl.BlockSpec((MBLK, 8), lambda i: (i, 0)),
          pl.BlockSpec((MBLK, 8), lambda i: (i, 0)),
      ],
      out_shape=[
          jax.ShapeDtypeStruct((NR8, 8), jnp.float32),
          jax.ShapeDtypeStruct((NR8, 8), jnp.float32),
          jax.ShapeDtypeStruct((NR8, 8), jnp.float32),
      ],
  )(o1p, b1p, KW, a2)


def _tc_post_body(o2_ref, pp_ref, out_ref):
  s = (o2_ref[0] + o2_ref[1] + pp_ref[0, 0]) * pp_ref[0, 1]
  out_ref[...] = jax.nn.sigmoid(s)


def _tc_post(o2p, pp):
  return pl.pallas_call(
      _tc_post_body,
      out_shape=jax.ShapeDtypeStruct((NR128, 128), jnp.float32),
  )(o2p, pp)


def _sc_att_body(src_ref, dst_ref, as_ref, ad_ref,
                 den_ref, ex_ref,
                 as_sh, ad_sh, den_sh,
                 src_v, dst_v, asg_v, adg_v, ex_v,
                 zb_v, sem, sem2):
  cid = lax.axis_index("c")
  sid = lax.axis_index("s")
  wid = sid * NC + cid
  sl = _stage_slice(sid)

  pltpu.sync_copy(as_ref.at[sl], zb_v)
  pltpu.sync_copy(zb_v, as_sh.at[sl])
  pltpu.sync_copy(ad_ref.at[sl], zb_v)
  pltpu.sync_copy(zb_v, ad_sh.at[sl])

  def _z(i, _):
    zb_v[pl.ds(i * 16, 16)] = _zero16()
    return 0
  lax.fori_loop(0, SL // 16, _z, 0)
  pltpu.sync_copy(zb_v, den_sh.at[sl])
  plsc.subcore_barrier()

  def _chunk(t, _):
    rbase = wid * W_ROWS + t * 8
    pltpu.sync_copy(src_ref.at[pl.ds(rbase, 8)], src_v)
    pltpu.sync_copy(dst_ref.at[pl.ds(rbase, 8)], dst_v)
    waits = []
    for i in range(8):
      waits.append(pltpu.async_copy(as_sh.at[src_v.at[i]], asg_v.at[i], sem))
      waits.append(pltpu.async_copy(ad_sh.at[dst_v.at[i]], adg_v.at[i], sem))
    for w in waits:
      w.wait()
    for r in range(8):
      for g in range(8):
        o = g * 16
        e = asg_v[r, pl.ds(o, 16)] + adg_v[r, pl.ds(o, 16)]
        e = jnp.where(e >= 0.0, e, 0.2 * e)
        ex_v[r, pl.ds(o, 16)] = jnp.exp(e)
    pltpu.sync_copy(ex_v, ex_ref.at[pl.ds(rbase, 8)])
    waits = []
    for i in range(8):
      waits.append(pltpu.async_copy(ex_v.at[i], den_sh.at[dst_v.at[i]], sem2,
                                    add=True))
    for w in waits:
      w.wait()
    return 0

  lax.fori_loop(0, W_ROWS // 8, _chunk, 0)
  plsc.subcore_barrier()
  pltpu.sync_copy(den_sh.at[sl], zb_v)
  pltpu.sync_copy(zb_v, den_ref.at[pl.ds(cid * N_PAD + sid * SL, SL)])


def _sc_att(src2, dst2, asp, adp):
  f = pl.kernel(
      _sc_att_body,
      out_type=[
          jax.ShapeDtypeStruct((NC * N_PAD,), jnp.float32),
          jax.ShapeDtypeStruct((ER, 128), jnp.float32),
      ],
      mesh=_mesh(),
      compiler_params=pltpu.CompilerParams(use_tc_tiling_on_sc=False),
      scratch_types=[
          pltpu.VMEM_SHARED((N_PAD,), jnp.float32),
          pltpu.VMEM_SHARED((N_PAD,), jnp.float32),
          pltpu.VMEM_SHARED((N_PAD,), jnp.float32),
          pltpu.VMEM((8, 128), jnp.int32),
          pltpu.VMEM((8, 128), jnp.int32),
          pltpu.VMEM((8, 128), jnp.float32),
          pltpu.VMEM((8, 128), jnp.float32),
          pltpu.VMEM((8, 128), jnp.float32),
          pltpu.VMEM((SL,), jnp.float32),
          pltpu.SemaphoreType.DMA,
          pltpu.SemaphoreType.DMA,
      ],
  )
  return f(src2, dst2, asp, adp)


def _sc_msg32_body(src_ref, dst_ref, ex_ref, ew_ref, denp_ref, h1s_ref,
                   out_ref,
                   den_sh, acc_sh,
                   src_v, dst_v, ex_v, ew_v, dg_v, w_v, rows_v,
                   pa_v, pb_v, zwb_v, sem, sem2, sem3):
  cid = lax.axis_index("c")
  sid = lax.axis_index("s")

  def _dcomb(k, _):
    off = sid * SL + k * WB_CH
    pltpu.sync_copy(denp_ref.at[pl.ds(off, WB_CH)], pa_v)
    pltpu.sync_copy(denp_ref.at[pl.ds(N_PAD + off, WB_CH)], pb_v)

    def _add(i, _):
      o = pl.ds(i * 16, 16)
      pa_v[o] = pa_v[o] + pb_v[o]
      return 0
    lax.fori_loop(0, WB_CH // 16, _add, 0)
    pltpu.sync_copy(pa_v, den_sh.at[pl.ds(off, WB_CH)])
    return 0
  lax.fori_loop(0, SL // WB_CH, _dcomb, 0)

  def _z(i, _):
    zwb_v[i] = _zero16()
    return 0
  lax.fori_loop(0, WB_CH, _z, 0)

  def _zc(k, _):
    pltpu.sync_copy(zwb_v, acc_sh.at[pl.ds(sid * SL + k * WB_CH, WB_CH), :])
    return 0
  lax.fori_loop(0, SL // WB_CH, _zc, 0)
  plsc.subcore_barrier()

  h_half = h1s_ref.at[cid]

  def _chunk(t, _):
    rbase = sid * S_ROWS + t * MCH
    pltpu.sync_copy(src_ref.at[pl.ds(rbase, MCH)], src_v)
    pltpu.sync_copy(dst_ref.at[pl.ds(rbase, MCH)], dst_v)
    pltpu.sync_copy(ex_ref.at[pl.ds(rbase, MCH)], ex_v)
    pltpu.sync_copy(ew_ref.at[pl.ds(rbase, MCH)], ew_v)
    gw, rw = [], []
    for i in range(MCH):
      gw.append(pltpu.async_copy(den_sh.at[dst_v.at[i]], dg_v.at[i], sem))
      rw.append(pltpu.async_copy(h_half.at[src_v.at[i]], rows_v.at[i], sem2))
    for w in gw:
      w.wait()

    def _wrow(r, _):
      for g in range(8):
        o = pl.ds(g * 16, 16)
        w_v[r, o] = ex_v[r, o] / (dg_v[r, o] + 1e-16) * ew_v[r, o]
      return 0
    lax.fori_loop(0, MCH, _wrow, 0)
    for w in rw:
      w.wait()

    def _scale_r(r, _):
      def _scale_c(g, _):
        wg = w_v[r, pl.ds(g * 16, 16)]
        for u in range(16):
          c = g * 16 + u
          rows_v[r, c] = rows_v[r, c] * jnp.broadcast_to(wg[u], (16,))
        return 0
      lax.fori_loop(0, 8, _scale_c, 0)
      return 0
    lax.fori_loop(0, MCH, _scale_r, 0)

    sw = []
    for i in range(MCH):
      sw.append(pltpu.async_copy(rows_v.at[i], acc_sh.at[dst_v.at[i]], sem3,
                                 add=True))
    for w in sw:
      w.wait()
    return 0

  lax.fori_loop(0, S_ROWS // MCH, _chunk, 0)
  plsc.subcore_barrier()

  def _wb(k, _):
    ro = sid * SL + k * WB_CH
    pltpu.sync_copy(acc_sh.at[pl.ds(ro, WB_CH), :], zwb_v)
    pltpu.sync_copy(zwb_v, out_ref.at[cid, pl.ds(ro, WB_CH), :])
    return 0
  lax.fori_loop(0, SL // WB_CH, _wb, 0)


def _sc_msg32(src2, dst2, ex2, ew2, den_part, h1s):
  f = pl.kernel(
      _sc_msg32_body,
      out_type=jax.ShapeDtypeStruct((2, N_PAD, HH), jnp.float32),
      mesh=_mesh(),
      compiler_params=pltpu.CompilerParams(use_tc_tiling_on_sc=False),
      scratch_types=[
          pltpu.VMEM_SHARED((N_PAD,), jnp.float32),
          pltpu.VMEM_SHARED((N_PAD, HH), jnp.float32),
          pltpu.VMEM((MCH, 128), jnp.int32),
          pltpu.VMEM((MCH, 128), jnp.int32),
          pltpu.VMEM((MCH, 128), jnp.float32),
          pltpu.VMEM((MCH, 128), jnp.float32),
          pltpu.VMEM((MCH, 128), jnp.float32),
          pltpu.VMEM((MCH, 128), jnp.float32),
          pltpu.VMEM((MCH, 128, HH), jnp.float32),
          pltpu.VMEM((WB_CH,), jnp.float32),
          pltpu.VMEM((WB_CH,), jnp.float32),
          pltpu.VMEM((WB_CH, HH), jnp.float32),
          pltpu.SemaphoreType.DMA,
          pltpu.SemaphoreType.DMA,
          pltpu.SemaphoreType.DMA,
      ],
  )
  return f(src2, dst2, ex2, ew2, den_part, h1s)


def _sc_msg1_body(src_ref, dst_ref, ex_ref, ew_ref, denp_ref, h2_ref,
                  out_ref,
                  den_sh, h2_sh, o2_sh,
                  src_v, dst_v, ex_v, ew_v, dg_v, hg_v, v_v,
                  pa_v, pb_v, zb_v, sem, sem2, sem3):
  cid = lax.axis_index("c")
  sid = lax.axis_index("s")
  wid = sid * NC + cid
  sl = _stage_slice(sid)

  pltpu.sync_copy(denp_ref.at[pl.ds(sid * SL, SL)], pa_v)
  pltpu.sync_copy(denp_ref.at[pl.ds(N_PAD + sid * SL, SL)], pb_v)

  def _add(i, _):
    o = pl.ds(i * 16, 16)
    pa_v[o] = pa_v[o] + pb_v[o]
    return 0
  lax.fori_loop(0, SL // 16, _add, 0)
  pltpu.sync_copy(pa_v, den_sh.at[sl])
  pltpu.sync_copy(h2_ref.at[sl], pb_v)
  pltpu.sync_copy(pb_v, h2_sh.at[sl])

  def _z(i, _):
    zb_v[pl.ds(i * 16, 16)] = _zero16()
    return 0
  lax.fori_loop(0, SL // 16, _z, 0)
  pltpu.sync_copy(zb_v, o2_sh.at[sl])
  plsc.subcore_barrier()

  def _chunk(t, _):
    rbase = wid * W_ROWS + t * 8
    pltpu.sync_copy(src_ref.at[pl.ds(rbase, 8)], src_v)
    pltpu.sync_copy(dst_ref.at[pl.ds(rbase, 8)], dst_v)
    pltpu.sync_copy(ex_ref.at[pl.ds(rbase, 8)], ex_v)
    pltpu.sync_copy(ew_ref.at[pl.ds(rbase, 8)], ew_v)
    waits = []
    for i in range(8):
      waits.append(pltpu.async_copy(den_sh.at[dst_v.at[i]], dg_v.at[i], sem))
      waits.append(pltpu.async_copy(h2_sh.at[src_v.at[i]], hg_v.at[i], sem2))
    for w in waits:
      w.wait()
    for r in range(8):
      for g in range(8):
        o = pl.ds(g * 16, 16)
        v_v[r, o] = ex_v[r, o] / (dg_v[r, o] + 1e-16) * ew_v[r, o] * hg_v[r, o]
    waits = []
    for i in range(8):
      waits.append(pltpu.async_copy(v_v.at[i], o2_sh.at[dst_v.at[i]], sem3,
                                    add=True))
    for w in waits:
      w.wait()
    return 0

  lax.fori_loop(0, W_ROWS // 8, _chunk, 0)
  plsc.subcore_barrier()
  pltpu.sync_copy(o2_sh.at[sl], pa_v)
  pltpu.sync_copy(pa_v, out_ref.at[pl.ds(cid * N_PAD + sid * SL, SL)])


def _sc_msg1(src2, dst2, ex2, ew2, den_part, h2p):
  f = pl.kernel(
      _sc_msg1_body,
      out_type=jax.ShapeDtypeStruct((NC * N_PAD,), jnp.float32),
      mesh=_mesh(),
      compiler_params=pltpu.CompilerParams(use_tc_tiling_on_sc=False),
      scratch_types=[
          pltpu.VMEM_SHARED((N_PAD,), jnp.float32),
          pltpu.VMEM_SHARED((N_PAD,), jnp.float32),
          pltpu.VMEM_SHARED((N_PAD,), jnp.float32),
          pltpu.VMEM((8, 128), jnp.int32),
          pltpu.VMEM((8, 128), jnp.int32),
          pltpu.VMEM((8, 128), jnp.float32),
          pltpu.VMEM((8, 128), jnp.float32),
          pltpu.VMEM((8, 128), jnp.float32),
          pltpu.VMEM((8, 128), jnp.float32),
          pltpu.VMEM((8, 128), jnp.float32),
          pltpu.VMEM((SL,), jnp.float32),
          pltpu.VMEM((SL,), jnp.float32),
          pltpu.VMEM((SL,), jnp.float32),
          pltpu.SemaphoreType.DMA,
          pltpu.SemaphoreType.DMA,
          pltpu.SemaphoreType.DMA,
      ],
  )
  return f(src2, dst2, ex2, ew2, den_part, h2p)


def kernel(x, edge_index, edge_attr, batch, W1, a_src1, a_dst1, b1,
           W2, a_src2, a_dst2, b2, Wl):
  del batch
  src = edge_index[0]
  dst = edge_index[1]
  ew = edge_attr[:, 0]

  npad = E_PAD - E
  spread = jnp.arange(npad, dtype=jnp.int32) % 8
  src2 = jnp.concatenate([src, spread]).reshape(ER, 128)
  dst2 = jnp.concatenate([dst, N + spread]).reshape(ER, 128)
  ew2 = jnp.concatenate([ew, jnp.zeros((npad,), jnp.float32)]).reshape(ER, 128)

  f32 = jnp.float32
  eye8 = jnp.eye(8, dtype=f32)
  xp = jnp.concatenate([x, jnp.zeros((N_PAD - N, x.shape[1]), f32)])
  xp = xp.reshape(NR8, 8 * x.shape[1])
  Kh0 = jnp.kron(eye8, W1[:, :HH])
  Kh1 = jnp.kron(eye8, W1[:, HH:])
  Kva = jnp.kron(eye8, (W1 @ a_src1)[:, None])
  Kvd = jnp.kron(eye8, (W1 @ a_dst1)[:, None])

  hpack, aspe, adpe = _tc_pre(xp, Kh0, Kh1, Kva, Kvd)
  h1s = hpack.reshape(2, N_PAD, HH)
  den1, ex1 = _sc_att(src2, dst2, aspe.reshape(N_PAD), adpe.reshape(N_PAD))
  out1s = _sc_msg32(src2, dst2, ex1, ew2, den1, h1s)

  o1p = out1s.reshape(2, NR8, 128)
  b1p = jnp.stack([jnp.tile(b1[:HH], 8), jnp.tile(b1[HH:], 8)])
  KW = jnp.stack([jnp.kron(eye8, W2[:HH]), jnp.kron(eye8, W2[HH:])])
  a2 = jnp.concatenate([a_src2, a_dst2]).reshape(1, 2)
  h2e, as2e, ad2e = _tc_mid(o1p, b1p, KW, a2)

  den2, ex2 = _sc_att(src2, dst2, as2e.reshape(N_PAD), ad2e.reshape(N_PAD))
  out2p = _sc_msg1(src2, dst2, ex2, ew2, den2, h2e.reshape(N_PAD))

  o2p = out2p.reshape(2, NR128, 128)
  pp = jnp.concatenate([b2, Wl[0]]).reshape(1, 2)
  y = _tc_post(o2p, pp)
  return y.reshape(N_PAD)[:N, None]
